# baseline (speedup 1.0000x reference)
import functools

import jax
import jax.numpy as jnp
from jax import lax
from jax.experimental import pallas as pl
from jax.experimental.pallas import tpu as pltpu
from jax.experimental.pallas import tpu_sc as plsc

B = 16384
D = 32


@functools.partial(jax.jit, static_argnums=())
def _sc_gather(user_table, item_table, user_idx, item_idx):
    info = plsc.get_sparse_core_info()
    nc, ns = info.num_cores, info.num_subcores
    nw = nc * ns
    bpw = B // nw

    mesh = plsc.VectorSubcoreMesh(core_axis_name="c", subcore_axis_name="s")

    @functools.partial(
        pl.kernel,
        mesh=mesh,
        out_type=[
            jax.ShapeDtypeStruct((B, D), jnp.float32),
            jax.ShapeDtypeStruct((B, D), jnp.float32),
        ],
        scratch_types=[
            pltpu.VMEM((bpw,), jnp.int32),
            pltpu.VMEM((bpw, D), jnp.float32),
            pltpu.VMEM((bpw,), jnp.int32),
            pltpu.VMEM((bpw, D), jnp.float32),
            pltpu.SemaphoreType.DMA,
        ],
        compiler_params=pltpu.CompilerParams(use_tc_tiling_on_sc=False),
    )
    def k(ut_hbm, it_hbm, ui_hbm, ii_hbm, uout_hbm, iout_hbm,
          uidx_v, urows_v, iidx_v, irows_v, sem):
        wid = lax.axis_index("s") * nc + lax.axis_index("c")
        base = wid * bpw
        pltpu.sync_copy(ui_hbm.at[pl.ds(base, bpw)], uidx_v)
        pltpu.sync_copy(ii_hbm.at[pl.ds(base, bpw)], iidx_v)
        cu = pltpu.async_copy(ut_hbm.at[uidx_v], urows_v, sem)
        ci = pltpu.async_copy(it_hbm.at[iidx_v], irows_v, sem)
        cu.wait()
        ci.wait()
        pltpu.sync_copy(urows_v, uout_hbm.at[pl.ds(base, bpw)])
        pltpu.sync_copy(irows_v, iout_hbm.at[pl.ds(base, bpw)])

    return k(user_table, item_table, user_idx, item_idx)


_BLK = 2048


def _mlp_body(u_ref, i_ref, d_ref, w1u_ref, w1i_ref, w1d_ref, b1_ref,
              w2_ref, b2_ref, w3_ref, b3_ref, o_ref):
    u = u_ref[...]
    it = i_ref[...]
    d = d_ref[...]
    h = jnp.dot(u, w1u_ref[...], preferred_element_type=jnp.float32)
    h += jnp.dot(it, w1i_ref[...], preferred_element_type=jnp.float32)
    h += d * w1d_ref[...]
    h = jnp.maximum(h + b1_ref[...], 0.0)
    h = jnp.dot(h, w2_ref[...], preferred_element_type=jnp.float32)
    h = jnp.maximum(h + b2_ref[...], 0.0)
    o_ref[...] = jnp.dot(h, w3_ref[...], preferred_element_type=jnp.float32) + b3_ref[...]


def _tc_mlp(u_emb, i_emb, dates, W1, b1, W2, b2, W3, b3):
    w1u = W1[:D]
    w1i = W1[D:2 * D]
    w1d = W1[2 * D:2 * D + 1]
    grid = (B // _BLK,)
    row_spec = lambda w: pl.BlockSpec((_BLK, w), lambda i: (i, 0))
    full = lambda a, b: pl.BlockSpec((a, b), lambda i: (0, 0))
    return pl.pallas_call(
        _mlp_body,
        grid=grid,
        in_specs=[
            row_spec(D),
            row_spec(D),
            row_spec(1),
            full(D, 64),
            full(D, 64),
            full(1, 64),
            full(1, 64),
            full(64, 32),
            full(1, 32),
            full(32, 2),
            full(1, 2),
        ],
        out_specs=pl.BlockSpec((_BLK, 2), lambda i: (i, 0)),
        out_shape=jax.ShapeDtypeStruct((B, 2), jnp.float32),
    )(u_emb, i_emb, dates, w1u, w1i, w1d, b1.reshape(1, 64),
      W2, b2.reshape(1, 32), W3, b3.reshape(1, 2))


def kernel(user_indices, item_indices, click_dates, user_table, item_table,
           W1, b1, W2, b2, W3, b3):
    ui = user_indices.astype(jnp.int32)
    ii = item_indices.astype(jnp.int32)
    u_emb, i_emb = _sc_gather(user_table, item_table, ui, ii)
    return _tc_mlp(u_emb, i_emb, click_dates, W1, b1, W2, b2, W3, b3)
